# baseline (speedup 1.0000x reference)
import functools

import jax
import jax.numpy as jnp
from jax import lax
from jax.experimental import pallas as pl
from jax.experimental.pallas import tpu as pltpu
from jax.experimental.pallas import tpu_sc as plsc

N = 10000
NPAD = 10240
D = 256
DH = 128
E = 160000
EPAD = 163840
NC = 2
NS = 16
C = 64
EPT = EPAD // NS
NCH = EPT // C
EPW = EPAD // (NC * NS)
NCHD = EPW // C
RPT = NPAD // NS
BN = 256
NB = NPAD // BN


def _deg_body(dstd_hbm, deg_hbm, dst_v, ones_v, zrow_v, deg_sh):
    c = lax.axis_index("c")
    s = lax.axis_index("s")
    w = c * NS + s
    pltpu.sync_copy(dstd_hbm.at[w], dst_v)
    for i in range(C // 16):
        ones_v[pl.ds(i * 16, 16)] = jnp.ones((16,), jnp.float32)
    for i in range(RPT // 16):
        zrow_v[pl.ds(i * 16, 16)] = jnp.zeros((16,), jnp.float32)
    pltpu.sync_copy(zrow_v, deg_sh.at[pl.ds(s * RPT, RPT)])
    plsc.subcore_barrier()

    def body(j, carry):
        pltpu.sync_copy(ones_v, deg_sh.at[dst_v.at[j]], add=True)
        return carry

    lax.fori_loop(0, NCHD, body, 0)
    plsc.subcore_barrier()
    pltpu.sync_copy(deg_sh.at[pl.ds(s * RPT, RPT)], deg_hbm.at[c, pl.ds(s * RPT, RPT)])


NBUF = 4
NISLOT = 8
NG = NCH // 8


def _edge_body(y_hbm, idx_hbm, acc0_hbm, acc1_hbm, idx_i, b0, b1, b2, b3,
               acc_sh, *sems):
    isems = sems[:NISLOT]
    gsems = sems[NISLOT:NISLOT + NBUF]
    ssems = sems[NISLOT + NBUF:]
    bufs = (b0, b1, b2, b3)
    c = lax.axis_index("c")
    s = lax.axis_index("s")
    pltpu.sync_copy(
        y_hbm.at[pl.ds(c * NPAD + s * RPT, RPT)],
        acc_sh.at[pl.ds(s * RPT, RPT)],
    )
    plsc.subcore_barrier()

    def idesc(j, u):
        return pltpu.make_async_copy(idx_hbm.at[c, s, j], idx_i.at[u], isems[u])

    def gdesc(j, u, b):
        del j
        return pltpu.make_async_copy(y_hbm.at[idx_i.at[u, 0]], bufs[b],
                                     gsems[b])

    def sdesc(j, u, b):
        del j
        return pltpu.make_async_copy(bufs[b], acc_sh.at[idx_i.at[u, 1]],
                                     ssems[b])

    for t in range(6):
        idesc(t, t).start()
    idesc(0, 0).wait()
    gdesc(0, 0, 0).start()
    idesc(1, 1).wait()
    gdesc(1, 1, 1).start()

    def body(g, carry):
        for u in range(8):
            j = g * 8 + u
            b = u % NBUF

            gdesc(j, u, b).wait()
            sdesc(j, u, b).start(add=True)

            @pl.when(j >= 2)
            def _():
                sdesc(j - 2, (u - 2) % 8, (u - 2) % 4).wait()

            @pl.when(j + 2 < NCH)
            def _():
                idesc(j + 2, (u + 2) % 8).wait()
                gdesc(j + 2, (u + 2) % 8, (u + 2) % 4).start()

            @pl.when(j + 6 < NCH)
            def _():
                idesc(j + 6, (u + 6) % 8).start()
        return carry

    lax.fori_loop(0, NG, body, 0)
    sdesc(NCH - 2, 6, 2).wait()
    sdesc(NCH - 1, 7, 3).wait()
    plsc.subcore_barrier()

    @pl.when(c == 0)
    def _():
        pltpu.sync_copy(acc_sh.at[pl.ds(s * RPT, RPT)],
                        acc0_hbm.at[pl.ds(s * RPT, RPT)])

    @pl.when(c == 1)
    def _():
        pltpu.sync_copy(acc_sh.at[pl.ds(s * RPT, RPT)],
                        acc1_hbm.at[pl.ds(s * RPT, RPT)])


def _sc_mesh():
    return plsc.VectorSubcoreMesh(core_axis_name="c", subcore_axis_name="s")


def _deg_call(dst_d):
    return pl.kernel(
        _deg_body,
        out_type=jax.ShapeDtypeStruct((NC, NPAD), jnp.float32),
        mesh=_sc_mesh(),
        scratch_types=[
            pltpu.VMEM((NCHD, C), jnp.int32),
            pltpu.VMEM((C,), jnp.float32),
            pltpu.VMEM((RPT,), jnp.float32),
            pltpu.VMEM_SHARED((NPAD,), jnp.float32),
        ],
    )(dst_d)


def _edge_call(y, idx_pair):
    return pl.kernel(
        _edge_body,
        out_type=(jax.ShapeDtypeStruct((NPAD, DH), jnp.float32),
                  jax.ShapeDtypeStruct((NPAD, DH), jnp.float32)),
        mesh=_sc_mesh(),
        scratch_types=[
            pltpu.VMEM((NISLOT, 2, C), jnp.int32),
            pltpu.VMEM((C, DH), jnp.float32),
            pltpu.VMEM((C, DH), jnp.float32),
            pltpu.VMEM((C, DH), jnp.float32),
            pltpu.VMEM((C, DH), jnp.float32),
            pltpu.VMEM_SHARED((NPAD, DH), jnp.float32),
        ] + [pltpu.SemaphoreType.DMA] * (NISLOT + 2 * NBUF),
    )(y, idx_pair)


def _tc_a_body(x_ref, w_ref, degp_ref, dk_ref, y_ref):
    dinv = lax.rsqrt(degp_ref[0, :] + degp_ref[1, :] + 1.0)
    xw = jnp.dot(x_ref[...] + dk_ref[0, 0], w_ref[...],
                 preferred_element_type=jnp.float32)
    y_ref[...] = xw * dinv[:, None]


def _tc_b_body(acc0_ref, acc1_ref, degp_ref, b1_ref, w2_ref, y2_ref):
    dinv = lax.rsqrt(degp_ref[0, :] + degp_ref[1, :] + 1.0)[:, None]
    h0 = jnp.maximum(acc0_ref[...] * dinv + b1_ref[0, :], 0.0)
    h1 = jnp.maximum(acc1_ref[...] * dinv + b1_ref[1, :], 0.0)
    y2 = (jnp.dot(h0, w2_ref[:DH, :], preferred_element_type=jnp.float32)
          + jnp.dot(h1, w2_ref[DH:, :], preferred_element_type=jnp.float32))
    y2_ref[...] = y2 * dinv


def _tc_c_body(acc0_ref, acc1_ref, degp_ref, b2_ref, out_ref):
    dinv = lax.rsqrt(degp_ref[0, :] + degp_ref[1, :] + 1.0)[:, None]
    h = jnp.concatenate([acc0_ref[...] * dinv, acc1_ref[...] * dinv], axis=1)
    h = jnp.maximum(h + b2_ref[0, :], 0.0)
    out_ref[...] = jnp.broadcast_to(h[:, None, :], (BN, 4, D))


def _tc_a(x_pad, w1, degp, dk):
    return pl.pallas_call(
        _tc_a_body,
        grid=(NB, NC),
        in_specs=[
            pl.BlockSpec((BN, D), lambda i, c: (i, 0)),
            pl.BlockSpec((D, DH), lambda i, c: (0, c)),
            pl.BlockSpec((NC, BN), lambda i, c: (0, i)),
            pl.BlockSpec((1, 1), lambda i, c: (0, 0)),
        ],
        out_specs=pl.BlockSpec((BN, DH), lambda i, c: (c * NB + i, 0)),
        out_shape=jax.ShapeDtypeStruct((NC * NPAD, DH), jnp.float32),
    )(x_pad, w1, degp, dk)


def _tc_b(acc0, acc1, degp, b1r, w2):
    return pl.pallas_call(
        _tc_b_body,
        grid=(NB, NC),
        in_specs=[
            pl.BlockSpec((BN, DH), lambda i, c: (i, 0)),
            pl.BlockSpec((BN, DH), lambda i, c: (i, 0)),
            pl.BlockSpec((NC, BN), lambda i, c: (0, i)),
            pl.BlockSpec((2, DH), lambda i, c: (0, 0)),
            pl.BlockSpec((D, DH), lambda i, c: (0, c)),
        ],
        out_specs=pl.BlockSpec((BN, DH), lambda i, c: (c * NB + i, 0)),
        out_shape=jax.ShapeDtypeStruct((NC * NPAD, DH), jnp.float32),
    )(acc0, acc1, degp, b1r, w2)


def _tc_c(acc0, acc1, degp, b2r):
    return pl.pallas_call(
        _tc_c_body,
        grid=(NB,),
        in_specs=[
            pl.BlockSpec((BN, DH), lambda i: (i, 0)),
            pl.BlockSpec((BN, DH), lambda i: (i, 0)),
            pl.BlockSpec((NC, BN), lambda i: (0, i)),
            pl.BlockSpec((1, D), lambda i: (0, 0)),
        ],
        out_specs=pl.BlockSpec((BN, 4, D), lambda i: (i, 0, 0)),
        out_shape=jax.ShapeDtypeStruct((N, 4, D), jnp.float32),
    )(acc0, acc1, degp, b2r)


def kernel(inputs, adj, W1, b1, W2, b2, K):
    src = adj[0].astype(jnp.int32)
    dst = adj[1].astype(jnp.int32)
    pad_idx = N + (jnp.arange(EPAD - E, dtype=jnp.int32) % (NPAD - N))
    src_p = jnp.concatenate([src, pad_idx])
    dst_p = jnp.concatenate([dst, pad_idx])
    srcs = jnp.stack([src_p, src_p + NPAD]).reshape(NC, NS, NCH, C)
    dsts = jnp.broadcast_to(dst_p.reshape(1, NS, NCH, C), (NC, NS, NCH, C))
    idx_pair = jnp.stack([srcs, dsts], axis=3)
    dst_d = dst_p.reshape(NC * NS, NCHD, C)

    dk = (jnp.asarray(K, jnp.float32) - 4.0).reshape(1, 1)
    b1r = b1.reshape(2, DH)
    b2r = b2.reshape(1, D)

    degp = _deg_call(dst_d)
    y1 = _tc_a(inputs, W1, degp, dk)
    acc1a, acc1b = _edge_call(y1, idx_pair)
    y2 = _tc_b(acc1a, acc1b, degp, b1r, W2)
    acc2a, acc2b = _edge_call(y2, idx_pair)
    return _tc_c(acc2a, acc2b, degp, b2r)


# speedup vs baseline: 1.0272x; 1.0272x over previous
import functools

import jax
import jax.numpy as jnp
from jax import lax
from jax.experimental import pallas as pl
from jax.experimental.pallas import tpu as pltpu
from jax.experimental.pallas import tpu_sc as plsc

N = 10000
NPAD = 10240
D = 256
DH = 128
E = 160000
EPAD = 163840
NC = 2
NS = 16
C = 80
EPT = EPAD // NS
NCH = EPT // C
EPW = EPAD // (NC * NS)
NCHD = EPW // C
RPT = NPAD // NS
BN = 256
NB = NPAD // BN


def _deg_body(dstd_hbm, deg_hbm, dst_v, ones_v, zrow_v, deg_sh):
    c = lax.axis_index("c")
    s = lax.axis_index("s")
    w = c * NS + s
    pltpu.sync_copy(dstd_hbm.at[w], dst_v)
    for i in range(C // 16):
        ones_v[pl.ds(i * 16, 16)] = jnp.ones((16,), jnp.float32)
    for i in range(RPT // 16):
        zrow_v[pl.ds(i * 16, 16)] = jnp.zeros((16,), jnp.float32)
    pltpu.sync_copy(zrow_v, deg_sh.at[pl.ds(s * RPT, RPT)])
    plsc.subcore_barrier()

    def body(j, carry):
        pltpu.sync_copy(ones_v, deg_sh.at[dst_v.at[j]], add=True)
        return carry

    lax.fori_loop(0, NCHD, body, 0)
    plsc.subcore_barrier()
    pltpu.sync_copy(deg_sh.at[pl.ds(s * RPT, RPT)], deg_hbm.at[c, pl.ds(s * RPT, RPT)])


NBUF = 4
NISLOT = 8
NG = NCH // 8


def _edge_body(y_hbm, idx_hbm, acc0_hbm, acc1_hbm, idx_i, b0, b1, b2, b3,
               acc_sh, *sems):
    isems = sems[:NISLOT]
    gsems = sems[NISLOT:NISLOT + NBUF]
    ssems = sems[NISLOT + NBUF:]
    bufs = (b0, b1, b2, b3)
    c = lax.axis_index("c")
    s = lax.axis_index("s")
    pltpu.sync_copy(
        y_hbm.at[pl.ds(c * NPAD + s * RPT, RPT)],
        acc_sh.at[pl.ds(s * RPT, RPT)],
    )
    plsc.subcore_barrier()

    def idesc(j, u):
        return pltpu.make_async_copy(idx_hbm.at[c, s, j], idx_i.at[u], isems[u])

    def gdesc(j, u, b):
        del j
        return pltpu.make_async_copy(y_hbm.at[idx_i.at[u, 0]], bufs[b],
                                     gsems[b])

    def sdesc(j, u, b):
        del j
        return pltpu.make_async_copy(bufs[b], acc_sh.at[idx_i.at[u, 1]],
                                     ssems[b])

    for t in range(6):
        idesc(t, t).start()
    idesc(0, 0).wait()
    gdesc(0, 0, 0).start()
    idesc(1, 1).wait()
    gdesc(1, 1, 1).start()

    def body(g, carry):
        for u in range(8):
            j = g * 8 + u
            b = u % NBUF

            gdesc(j, u, b).wait()
            sdesc(j, u, b).start(add=True)

            @pl.when(j >= 2)
            def _():
                sdesc(j - 2, (u - 2) % 8, (u - 2) % 4).wait()

            @pl.when(j + 2 < NCH)
            def _():
                idesc(j + 2, (u + 2) % 8).wait()
                gdesc(j + 2, (u + 2) % 8, (u + 2) % 4).start()

            @pl.when(j + 6 < NCH)
            def _():
                idesc(j + 6, (u + 6) % 8).start()
        return carry

    lax.fori_loop(0, NG, body, 0)
    sdesc(NCH - 2, 6, 2).wait()
    sdesc(NCH - 1, 7, 3).wait()
    plsc.subcore_barrier()

    @pl.when(c == 0)
    def _():
        pltpu.sync_copy(acc_sh.at[pl.ds(s * RPT, RPT)],
                        acc0_hbm.at[pl.ds(s * RPT, RPT)])

    @pl.when(c == 1)
    def _():
        pltpu.sync_copy(acc_sh.at[pl.ds(s * RPT, RPT)],
                        acc1_hbm.at[pl.ds(s * RPT, RPT)])


def _sc_mesh():
    return plsc.VectorSubcoreMesh(core_axis_name="c", subcore_axis_name="s")


def _deg_call(dst_d):
    return pl.kernel(
        _deg_body,
        out_type=jax.ShapeDtypeStruct((NC, NPAD), jnp.float32),
        mesh=_sc_mesh(),
        scratch_types=[
            pltpu.VMEM((NCHD, C), jnp.int32),
            pltpu.VMEM((C,), jnp.float32),
            pltpu.VMEM((RPT,), jnp.float32),
            pltpu.VMEM_SHARED((NPAD,), jnp.float32),
        ],
    )(dst_d)


def _edge_call(y, idx_pair):
    return pl.kernel(
        _edge_body,
        out_type=(jax.ShapeDtypeStruct((NPAD, DH), jnp.float32),
                  jax.ShapeDtypeStruct((NPAD, DH), jnp.float32)),
        mesh=_sc_mesh(),
        scratch_types=[
            pltpu.VMEM((NISLOT, 2, C), jnp.int32),
            pltpu.VMEM((C, DH), jnp.float32),
            pltpu.VMEM((C, DH), jnp.float32),
            pltpu.VMEM((C, DH), jnp.float32),
            pltpu.VMEM((C, DH), jnp.float32),
            pltpu.VMEM_SHARED((NPAD, DH), jnp.float32),
        ] + [pltpu.SemaphoreType.DMA] * (NISLOT + 2 * NBUF),
    )(y, idx_pair)


def _tc_a_body(x_ref, w_ref, degp_ref, dk_ref, y_ref):
    dinv = lax.rsqrt(degp_ref[0, :] + degp_ref[1, :] + 1.0)
    xw = jnp.dot(x_ref[...] + dk_ref[0, 0], w_ref[...],
                 preferred_element_type=jnp.float32)
    y_ref[...] = xw * dinv[:, None]


def _tc_b_body(acc0_ref, acc1_ref, degp_ref, b1_ref, w2_ref, y2_ref):
    dinv = lax.rsqrt(degp_ref[0, :] + degp_ref[1, :] + 1.0)[:, None]
    h0 = jnp.maximum(acc0_ref[...] * dinv + b1_ref[0, :], 0.0)
    h1 = jnp.maximum(acc1_ref[...] * dinv + b1_ref[1, :], 0.0)
    y2 = (jnp.dot(h0, w2_ref[:DH, :], preferred_element_type=jnp.float32)
          + jnp.dot(h1, w2_ref[DH:, :], preferred_element_type=jnp.float32))
    y2_ref[...] = y2 * dinv


def _tc_c_body(acc0_ref, acc1_ref, degp_ref, b2_ref, out_ref):
    dinv = lax.rsqrt(degp_ref[0, :] + degp_ref[1, :] + 1.0)[:, None]
    h = jnp.concatenate([acc0_ref[...] * dinv, acc1_ref[...] * dinv], axis=1)
    h = jnp.maximum(h + b2_ref[0, :], 0.0)
    out_ref[...] = jnp.broadcast_to(h[:, None, :], (BN, 4, D))


def _tc_a(x_pad, w1, degp, dk):
    return pl.pallas_call(
        _tc_a_body,
        grid=(NB, NC),
        in_specs=[
            pl.BlockSpec((BN, D), lambda i, c: (i, 0)),
            pl.BlockSpec((D, DH), lambda i, c: (0, c)),
            pl.BlockSpec((NC, BN), lambda i, c: (0, i)),
            pl.BlockSpec((1, 1), lambda i, c: (0, 0)),
        ],
        out_specs=pl.BlockSpec((BN, DH), lambda i, c: (c * NB + i, 0)),
        out_shape=jax.ShapeDtypeStruct((NC * NPAD, DH), jnp.float32),
    )(x_pad, w1, degp, dk)


def _tc_b(acc0, acc1, degp, b1r, w2):
    return pl.pallas_call(
        _tc_b_body,
        grid=(NB, NC),
        in_specs=[
            pl.BlockSpec((BN, DH), lambda i, c: (i, 0)),
            pl.BlockSpec((BN, DH), lambda i, c: (i, 0)),
            pl.BlockSpec((NC, BN), lambda i, c: (0, i)),
            pl.BlockSpec((2, DH), lambda i, c: (0, 0)),
            pl.BlockSpec((D, DH), lambda i, c: (0, c)),
        ],
        out_specs=pl.BlockSpec((BN, DH), lambda i, c: (c * NB + i, 0)),
        out_shape=jax.ShapeDtypeStruct((NC * NPAD, DH), jnp.float32),
    )(acc0, acc1, degp, b1r, w2)


def _tc_c(acc0, acc1, degp, b2r):
    return pl.pallas_call(
        _tc_c_body,
        grid=(NB,),
        in_specs=[
            pl.BlockSpec((BN, DH), lambda i: (i, 0)),
            pl.BlockSpec((BN, DH), lambda i: (i, 0)),
            pl.BlockSpec((NC, BN), lambda i: (0, i)),
            pl.BlockSpec((1, D), lambda i: (0, 0)),
        ],
        out_specs=pl.BlockSpec((BN, 4, D), lambda i: (i, 0, 0)),
        out_shape=jax.ShapeDtypeStruct((N, 4, D), jnp.float32),
    )(acc0, acc1, degp, b2r)


def kernel(inputs, adj, W1, b1, W2, b2, K):
    src = adj[0].astype(jnp.int32)
    dst = adj[1].astype(jnp.int32)
    pad_idx = N + (jnp.arange(EPAD - E, dtype=jnp.int32) % (NPAD - N))
    src_p = jnp.concatenate([src, pad_idx])
    dst_p = jnp.concatenate([dst, pad_idx])
    srcs = jnp.stack([src_p, src_p + NPAD]).reshape(NC, NS, NCH, C)
    dsts = jnp.broadcast_to(dst_p.reshape(1, NS, NCH, C), (NC, NS, NCH, C))
    idx_pair = jnp.stack([srcs, dsts], axis=3)
    dst_d = dst_p.reshape(NC * NS, NCHD, C)

    dk = (jnp.asarray(K, jnp.float32) - 4.0).reshape(1, 1)
    b1r = b1.reshape(2, DH)
    b2r = b2.reshape(1, D)

    degp = _deg_call(dst_d)
    y1 = _tc_a(inputs, W1, degp, dk)
    acc1a, acc1b = _edge_call(y1, idx_pair)
    y2 = _tc_b(acc1a, acc1b, degp, b1r, W2)
    acc2a, acc2b = _edge_call(y2, idx_pair)
    return _tc_c(acc2a, acc2b, degp, b2r)


# speedup vs baseline: 1.2724x; 1.2387x over previous
import functools

import jax
import jax.numpy as jnp
from jax import lax
from jax.experimental import pallas as pl
from jax.experimental.pallas import tpu as pltpu
from jax.experimental.pallas import tpu_sc as plsc

N = 10000
NPAD = 10240
D = 256
DH = 128
E = 160000
EPAD = 163840
NC = 2
NS = 16
C = 128
EPT = EPAD // NS
NCH = EPT // C
EPW = EPAD // (NC * NS)
NCHD = EPW // C
RPT = NPAD // NS
BN = 256
NB = NPAD // BN


def _deg_body(dstd_hbm, deg_hbm, dst_v, ones_v, zrow_v, deg_sh):
    c = lax.axis_index("c")
    s = lax.axis_index("s")
    w = c * NS + s
    pltpu.sync_copy(dstd_hbm.at[w], dst_v)
    for i in range(C // 16):
        ones_v[pl.ds(i * 16, 16)] = jnp.ones((16,), jnp.float32)
    for i in range(RPT // 16):
        zrow_v[pl.ds(i * 16, 16)] = jnp.zeros((16,), jnp.float32)
    pltpu.sync_copy(zrow_v, deg_sh.at[pl.ds(s * RPT, RPT)])
    plsc.subcore_barrier()

    def body(j, carry):
        pltpu.sync_copy(ones_v, deg_sh.at[dst_v.at[j]], add=True)
        return carry

    lax.fori_loop(0, NCHD, body, 0)
    plsc.subcore_barrier()
    pltpu.sync_copy(deg_sh.at[pl.ds(s * RPT, RPT)], deg_hbm.at[c, pl.ds(s * RPT, RPT)])


NISLOT = 4
NG = NCH // 4


def _edge_body(y_hbm, idx_hbm, acc0_hbm, acc1_hbm, idx_i, buf0, buf1, acc_sh,
               *sems):
    isems = sems[:NISLOT]
    gsems = sems[NISLOT:NISLOT + 2]
    ssems = sems[NISLOT + 2:]
    bufs = (buf0, buf1)
    c = lax.axis_index("c")
    s = lax.axis_index("s")
    pltpu.sync_copy(
        y_hbm.at[pl.ds(c * NPAD + s * RPT, RPT)],
        acc_sh.at[pl.ds(s * RPT, RPT)],
    )
    plsc.subcore_barrier()

    def idesc(j, u):
        return pltpu.make_async_copy(idx_hbm.at[c, s, j], idx_i.at[u], isems[u])

    def gdesc(j, u, b):
        del j
        return pltpu.make_async_copy(y_hbm.at[idx_i.at[u, 0]], bufs[b], gsems[b])

    def sdesc(j, u, b):
        del j
        return pltpu.make_async_copy(bufs[b], acc_sh.at[idx_i.at[u, 1]], ssems[b])

    for t in range(3):
        idesc(t, t).start()
    idesc(0, 0).wait()
    gdesc(0, 0, 0).start()

    def body(g, carry):
        for u in range(4):
            j = g * 4 + u
            b = u % 2

            @pl.when(j >= 1)
            def _():
                sdesc(j - 1, (u - 1) % 4, 1 - b).wait()

            @pl.when(j + 1 < NCH)
            def _():
                idesc(j + 1, (u + 1) % 4).wait()
                gdesc(j + 1, (u + 1) % 4, 1 - b).start()

            gdesc(j, u, b).wait()
            sdesc(j, u, b).start(add=True)

            @pl.when(j + 3 < NCH)
            def _():
                idesc(j + 3, (u + 3) % 4).start()
        return carry

    lax.fori_loop(0, NG, body, 0)
    sdesc(NCH - 1, 3, 1).wait()
    plsc.subcore_barrier()

    @pl.when(c == 0)
    def _():
        pltpu.sync_copy(acc_sh.at[pl.ds(s * RPT, RPT)],
                        acc0_hbm.at[pl.ds(s * RPT, RPT)])

    @pl.when(c == 1)
    def _():
        pltpu.sync_copy(acc_sh.at[pl.ds(s * RPT, RPT)],
                        acc1_hbm.at[pl.ds(s * RPT, RPT)])


def _sc_mesh():
    return plsc.VectorSubcoreMesh(core_axis_name="c", subcore_axis_name="s")


def _deg_call(dst_d):
    return pl.kernel(
        _deg_body,
        out_type=jax.ShapeDtypeStruct((NC, NPAD), jnp.float32),
        mesh=_sc_mesh(),
        scratch_types=[
            pltpu.VMEM((NCHD, C), jnp.int32),
            pltpu.VMEM((C,), jnp.float32),
            pltpu.VMEM((RPT,), jnp.float32),
            pltpu.VMEM_SHARED((NPAD,), jnp.float32),
        ],
    )(dst_d)


def _edge_call(y, idx_pair):
    return pl.kernel(
        _edge_body,
        out_type=(jax.ShapeDtypeStruct((NPAD, DH), jnp.float32),
                  jax.ShapeDtypeStruct((NPAD, DH), jnp.float32)),
        mesh=_sc_mesh(),
        scratch_types=[
            pltpu.VMEM((NISLOT, 2, C), jnp.int32),
            pltpu.VMEM((C, DH), jnp.float32),
            pltpu.VMEM((C, DH), jnp.float32),
            pltpu.VMEM_SHARED((NPAD, DH), jnp.float32),
        ] + [pltpu.SemaphoreType.DMA] * (NISLOT + 4),
    )(y, idx_pair)


def _tc_a_body(x_ref, w_ref, degp_ref, dk_ref, y_ref):
    dinv = lax.rsqrt(degp_ref[0, :] + degp_ref[1, :] + 1.0)[:, None]
    xw = jnp.dot(x_ref[...] + dk_ref[0, 0], w_ref[...],
                 preferred_element_type=jnp.float32)
    y = xw * dinv
    y_ref[0] = y[:, :DH]
    y_ref[1] = y[:, DH:]


def _tc_b_body(acc0_ref, acc1_ref, degp_ref, b1_ref, w2_ref, y2_ref):
    dinv = lax.rsqrt(degp_ref[0, :] + degp_ref[1, :] + 1.0)[:, None]
    h0 = jnp.maximum(acc0_ref[...] * dinv + b1_ref[0, :], 0.0)
    h1 = jnp.maximum(acc1_ref[...] * dinv + b1_ref[1, :], 0.0)
    y2 = (jnp.dot(h0, w2_ref[:DH, :], preferred_element_type=jnp.float32)
          + jnp.dot(h1, w2_ref[DH:, :], preferred_element_type=jnp.float32))
    y2 = y2 * dinv
    y2_ref[0] = y2[:, :DH]
    y2_ref[1] = y2[:, DH:]


def _tc_c_body(acc0_ref, acc1_ref, degp_ref, b2_ref, out_ref):
    dinv = lax.rsqrt(degp_ref[0, :] + degp_ref[1, :] + 1.0)[:, None]
    h = jnp.concatenate([acc0_ref[...] * dinv, acc1_ref[...] * dinv], axis=1)
    h = jnp.maximum(h + b2_ref[0, :], 0.0)
    out_ref[...] = jnp.broadcast_to(h[:, None, :], (BN, 4, D))


def _tc_a(x, w1, degp, dk):
    return pl.pallas_call(
        _tc_a_body,
        grid=(NB,),
        in_specs=[
            pl.BlockSpec((BN, D), lambda i: (i, 0)),
            pl.BlockSpec((D, D), lambda i: (0, 0)),
            pl.BlockSpec((NC, BN), lambda i: (0, i)),
            pl.BlockSpec((1, 1), lambda i: (0, 0)),
        ],
        out_specs=pl.BlockSpec((NC, BN, DH), lambda i: (0, i, 0)),
        out_shape=jax.ShapeDtypeStruct((NC, NPAD, DH), jnp.float32),
    )(x, w1, degp, dk)


def _tc_b(acc0, acc1, degp, b1r, w2):
    return pl.pallas_call(
        _tc_b_body,
        grid=(NB,),
        in_specs=[
            pl.BlockSpec((BN, DH), lambda i: (i, 0)),
            pl.BlockSpec((BN, DH), lambda i: (i, 0)),
            pl.BlockSpec((NC, BN), lambda i: (0, i)),
            pl.BlockSpec((2, DH), lambda i: (0, 0)),
            pl.BlockSpec((D, D), lambda i: (0, 0)),
        ],
        out_specs=pl.BlockSpec((NC, BN, DH), lambda i: (0, i, 0)),
        out_shape=jax.ShapeDtypeStruct((NC, NPAD, DH), jnp.float32),
    )(acc0, acc1, degp, b1r, w2)


def _tc_c(acc0, acc1, degp, b2r):
    return pl.pallas_call(
        _tc_c_body,
        grid=(NB,),
        in_specs=[
            pl.BlockSpec((BN, DH), lambda i: (i, 0)),
            pl.BlockSpec((BN, DH), lambda i: (i, 0)),
            pl.BlockSpec((NC, BN), lambda i: (0, i)),
            pl.BlockSpec((1, D), lambda i: (0, 0)),
        ],
        out_specs=pl.BlockSpec((BN, 4, D), lambda i: (i, 0, 0)),
        out_shape=jax.ShapeDtypeStruct((N, 4, D), jnp.float32),
    )(acc0, acc1, degp, b2r)


def kernel(inputs, adj, W1, b1, W2, b2, K):
    src = adj[0].astype(jnp.int32)
    dst = adj[1].astype(jnp.int32)
    pad_idx = N + (jnp.arange(EPAD - E, dtype=jnp.int32) % (NPAD - N))
    src_p = jnp.concatenate([src, pad_idx])
    dst_p = jnp.concatenate([dst, pad_idx])
    srcs = jnp.stack([src_p, src_p + NPAD]).reshape(NC, NS, NCH, C)
    dsts = jnp.broadcast_to(dst_p.reshape(1, NS, NCH, C), (NC, NS, NCH, C))
    idx_pair = jnp.stack([srcs, dsts], axis=3)
    dst_d = dst_p.reshape(NC * NS, NCHD, C)

    dk = (jnp.asarray(K, jnp.float32) - 4.0).reshape(1, 1)
    b1r = b1.reshape(2, DH)
    b2r = b2.reshape(1, D)

    degp = _deg_call(dst_d)
    y1 = _tc_a(inputs, W1, degp, dk).reshape(NC * NPAD, DH)
    acc1a, acc1b = _edge_call(y1, idx_pair)
    y2 = _tc_b(acc1a, acc1b, degp, b1r, W2).reshape(NC * NPAD, DH)
    acc2a, acc2b = _edge_call(y2, idx_pair)
    return _tc_c(acc2a, acc2b, degp, b2r)


# speedup vs baseline: 1.4028x; 1.1024x over previous
import functools

import jax
import jax.numpy as jnp
from jax import lax
from jax.experimental import pallas as pl
from jax.experimental.pallas import tpu as pltpu
from jax.experimental.pallas import tpu_sc as plsc

N = 10000
NPAD = 10240
D = 256
DH = 128
E = 160000
EPAD = 163840
NC = 2
NS = 16
C = 128
EPT = EPAD // NS
NCH = EPT // C
EPW = EPAD // (NC * NS)
NCHD = EPW // C
RPT = NPAD // NS
BN = 512
NB = NPAD // BN


def _deg_body(dstd_hbm, deg_hbm, dst_v, ones_v, zrow_v, deg_sh):
    c = lax.axis_index("c")
    s = lax.axis_index("s")
    w = c * NS + s
    pltpu.sync_copy(dstd_hbm.at[w], dst_v)
    for i in range(C // 16):
        ones_v[pl.ds(i * 16, 16)] = jnp.ones((16,), jnp.float32)
    for i in range(RPT // 16):
        zrow_v[pl.ds(i * 16, 16)] = jnp.zeros((16,), jnp.float32)
    pltpu.sync_copy(zrow_v, deg_sh.at[pl.ds(s * RPT, RPT)])
    plsc.subcore_barrier()

    def body(j, carry):
        pltpu.sync_copy(ones_v, deg_sh.at[dst_v.at[j]], add=True)
        return carry

    lax.fori_loop(0, NCHD, body, 0)
    plsc.subcore_barrier()
    pltpu.sync_copy(deg_sh.at[pl.ds(s * RPT, RPT)], deg_hbm.at[c, pl.ds(s * RPT, RPT)])


NISLOT = 4
NG = NCH // 4


def _edge_body(y_hbm, idx_hbm, acc0_hbm, acc1_hbm, idx_i, buf0, buf1, acc_sh,
               *sems):
    isems = sems[:NISLOT]
    gsems = sems[NISLOT:NISLOT + 2]
    ssems = sems[NISLOT + 2:]
    bufs = (buf0, buf1)
    c = lax.axis_index("c")
    s = lax.axis_index("s")
    pltpu.sync_copy(
        y_hbm.at[pl.ds(c * NPAD + s * RPT, RPT)],
        acc_sh.at[pl.ds(s * RPT, RPT)],
    )
    plsc.subcore_barrier()

    def idesc(j, u):
        return pltpu.make_async_copy(idx_hbm.at[c, s, j], idx_i.at[u], isems[u])

    def gdesc(j, u, b):
        del j
        return pltpu.make_async_copy(y_hbm.at[idx_i.at[u, 0]], bufs[b], gsems[b])

    def sdesc(j, u, b):
        del j
        return pltpu.make_async_copy(bufs[b], acc_sh.at[idx_i.at[u, 1]], ssems[b])

    for t in range(3):
        idesc(t, t).start()
    idesc(0, 0).wait()
    gdesc(0, 0, 0).start()

    def body(g, carry):
        for u in range(4):
            j = g * 4 + u
            b = u % 2

            @pl.when(j >= 1)
            def _():
                sdesc(j - 1, (u - 1) % 4, 1 - b).wait()

            @pl.when(j + 1 < NCH)
            def _():
                idesc(j + 1, (u + 1) % 4).wait()
                gdesc(j + 1, (u + 1) % 4, 1 - b).start()

            gdesc(j, u, b).wait()
            sdesc(j, u, b).start(add=True)

            @pl.when(j + 3 < NCH)
            def _():
                idesc(j + 3, (u + 3) % 4).start()
        return carry

    lax.fori_loop(0, NG, body, 0)
    sdesc(NCH - 1, 3, 1).wait()
    plsc.subcore_barrier()

    @pl.when(c == 0)
    def _():
        pltpu.sync_copy(acc_sh.at[pl.ds(s * RPT, RPT)],
                        acc0_hbm.at[pl.ds(s * RPT, RPT)])

    @pl.when(c == 1)
    def _():
        pltpu.sync_copy(acc_sh.at[pl.ds(s * RPT, RPT)],
                        acc1_hbm.at[pl.ds(s * RPT, RPT)])


def _sc_mesh():
    return plsc.VectorSubcoreMesh(core_axis_name="c", subcore_axis_name="s")


def _deg_call(dst_d):
    return pl.kernel(
        _deg_body,
        out_type=jax.ShapeDtypeStruct((NC, NPAD), jnp.float32),
        mesh=_sc_mesh(),
        scratch_types=[
            pltpu.VMEM((NCHD, C), jnp.int32),
            pltpu.VMEM((C,), jnp.float32),
            pltpu.VMEM((RPT,), jnp.float32),
            pltpu.VMEM_SHARED((NPAD,), jnp.float32),
        ],
    )(dst_d)


def _edge_call(y, idx_pair):
    return pl.kernel(
        _edge_body,
        out_type=(jax.ShapeDtypeStruct((NPAD, DH), jnp.float32),
                  jax.ShapeDtypeStruct((NPAD, DH), jnp.float32)),
        mesh=_sc_mesh(),
        scratch_types=[
            pltpu.VMEM((NISLOT, 2, C), jnp.int32),
            pltpu.VMEM((C, DH), jnp.float32),
            pltpu.VMEM((C, DH), jnp.float32),
            pltpu.VMEM_SHARED((NPAD, DH), jnp.float32),
        ] + [pltpu.SemaphoreType.DMA] * (NISLOT + 4),
    )(y, idx_pair)


def _tc_a_body(x_ref, w_ref, degp_ref, dk_ref, y_ref):
    dinv = lax.rsqrt(degp_ref[0, :] + degp_ref[1, :] + 1.0)[:, None]
    xw = jnp.dot(x_ref[...] + dk_ref[0, 0], w_ref[...],
                 preferred_element_type=jnp.float32)
    y = xw * dinv
    y_ref[0] = y[:, :DH]
    y_ref[1] = y[:, DH:]


def _tc_b_body(acc0_ref, acc1_ref, degp_ref, b1_ref, w2_ref, y2_ref):
    dinv = lax.rsqrt(degp_ref[0, :] + degp_ref[1, :] + 1.0)[:, None]
    h0 = jnp.maximum(acc0_ref[...] * dinv + b1_ref[0, :], 0.0)
    h1 = jnp.maximum(acc1_ref[...] * dinv + b1_ref[1, :], 0.0)
    y2 = (jnp.dot(h0, w2_ref[:DH, :], preferred_element_type=jnp.float32)
          + jnp.dot(h1, w2_ref[DH:, :], preferred_element_type=jnp.float32))
    y2 = y2 * dinv
    y2_ref[0] = y2[:, :DH]
    y2_ref[1] = y2[:, DH:]


def _tc_c_body(acc0_ref, acc1_ref, degp_ref, b2_ref, out_ref):
    dinv = lax.rsqrt(degp_ref[0, :] + degp_ref[1, :] + 1.0)[:, None]
    h = jnp.concatenate([acc0_ref[...] * dinv, acc1_ref[...] * dinv], axis=1)
    h = jnp.maximum(h + b2_ref[0, :], 0.0)
    out_ref[...] = jnp.broadcast_to(h[:, None, :], (BN, 4, D))


def _tc_a(x, w1, degp, dk):
    return pl.pallas_call(
        _tc_a_body,
        grid=(NB,),
        in_specs=[
            pl.BlockSpec((BN, D), lambda i: (i, 0)),
            pl.BlockSpec((D, D), lambda i: (0, 0)),
            pl.BlockSpec((NC, BN), lambda i: (0, i)),
            pl.BlockSpec((1, 1), lambda i: (0, 0)),
        ],
        out_specs=pl.BlockSpec((NC, BN, DH), lambda i: (0, i, 0)),
        out_shape=jax.ShapeDtypeStruct((NC, NPAD, DH), jnp.float32),
    )(x, w1, degp, dk)


def _tc_b(acc0, acc1, degp, b1r, w2):
    return pl.pallas_call(
        _tc_b_body,
        grid=(NB,),
        in_specs=[
            pl.BlockSpec((BN, DH), lambda i: (i, 0)),
            pl.BlockSpec((BN, DH), lambda i: (i, 0)),
            pl.BlockSpec((NC, BN), lambda i: (0, i)),
            pl.BlockSpec((2, DH), lambda i: (0, 0)),
            pl.BlockSpec((D, D), lambda i: (0, 0)),
        ],
        out_specs=pl.BlockSpec((NC, BN, DH), lambda i: (0, i, 0)),
        out_shape=jax.ShapeDtypeStruct((NC, NPAD, DH), jnp.float32),
    )(acc0, acc1, degp, b1r, w2)


def _tc_c(acc0, acc1, degp, b2r):
    return pl.pallas_call(
        _tc_c_body,
        grid=(NB,),
        in_specs=[
            pl.BlockSpec((BN, DH), lambda i: (i, 0)),
            pl.BlockSpec((BN, DH), lambda i: (i, 0)),
            pl.BlockSpec((NC, BN), lambda i: (0, i)),
            pl.BlockSpec((1, D), lambda i: (0, 0)),
        ],
        out_specs=pl.BlockSpec((BN, 4, D), lambda i: (i, 0, 0)),
        out_shape=jax.ShapeDtypeStruct((N, 4, D), jnp.float32),
    )(acc0, acc1, degp, b2r)


def kernel(inputs, adj, W1, b1, W2, b2, K):
    src = adj[0].astype(jnp.int32)
    dst = adj[1].astype(jnp.int32)
    pad_idx = N + (jnp.arange(EPAD - E, dtype=jnp.int32) % (NPAD - N))
    src_p = jnp.concatenate([src, pad_idx])
    dst_p = jnp.concatenate([dst, pad_idx])
    srcs = jnp.stack([src_p, src_p + NPAD]).reshape(NC, NS, NCH, C)
    dsts = jnp.broadcast_to(dst_p.reshape(1, NS, NCH, C), (NC, NS, NCH, C))
    idx_pair = jnp.stack([srcs, dsts], axis=3)
    dst_d = dst_p.reshape(NC * NS, NCHD, C)

    dk = (jnp.asarray(K, jnp.float32) - 4.0).reshape(1, 1)
    b1r = b1.reshape(2, DH)
    b2r = b2.reshape(1, D)

    degp = _deg_call(dst_d)
    y1 = _tc_a(inputs, W1, degp, dk).reshape(NC * NPAD, DH)
    acc1a, acc1b = _edge_call(y1, idx_pair)
    y2 = _tc_b(acc1a, acc1b, degp, b1r, W2).reshape(NC * NPAD, DH)
    acc2a, acc2b = _edge_call(y2, idx_pair)
    return _tc_c(acc2a, acc2b, degp, b2r)


# speedup vs baseline: 1.4725x; 1.0497x over previous
import functools

import jax
import jax.numpy as jnp
from jax import lax
from jax.experimental import pallas as pl
from jax.experimental.pallas import tpu as pltpu
from jax.experimental.pallas import tpu_sc as plsc

N = 10000
NPAD = 10240
D = 256
DH = 128
E = 160000
EPAD = 163840
NC = 2
NS = 16
C = 128
EPT = EPAD // NS
NCH = EPT // C
EPW = EPAD // (NC * NS)
NCHD = EPW // C
RPT = NPAD // NS
BN = 1024
NB = NPAD // BN


def _deg_body(dstd_hbm, deg_hbm, dst_v, ones_v, zrow_v, deg_sh):
    c = lax.axis_index("c")
    s = lax.axis_index("s")
    w = c * NS + s
    pltpu.sync_copy(dstd_hbm.at[w], dst_v)
    for i in range(C // 16):
        ones_v[pl.ds(i * 16, 16)] = jnp.ones((16,), jnp.float32)
    for i in range(RPT // 16):
        zrow_v[pl.ds(i * 16, 16)] = jnp.zeros((16,), jnp.float32)
    pltpu.sync_copy(zrow_v, deg_sh.at[pl.ds(s * RPT, RPT)])
    plsc.subcore_barrier()

    def body(j, carry):
        pltpu.sync_copy(ones_v, deg_sh.at[dst_v.at[j]], add=True)
        return carry

    lax.fori_loop(0, NCHD, body, 0)
    plsc.subcore_barrier()
    pltpu.sync_copy(deg_sh.at[pl.ds(s * RPT, RPT)], deg_hbm.at[c, pl.ds(s * RPT, RPT)])


NISLOT = 4
NG = NCH // 4


def _edge_body(y_hbm, idx_hbm, acc0_hbm, acc1_hbm, idx_i, buf0, buf1, acc_sh,
               *sems):
    isems = sems[:NISLOT]
    gsems = sems[NISLOT:NISLOT + 2]
    ssems = sems[NISLOT + 2:]
    bufs = (buf0, buf1)
    c = lax.axis_index("c")
    s = lax.axis_index("s")
    pltpu.sync_copy(
        y_hbm.at[pl.ds(c * NPAD + s * RPT, RPT)],
        acc_sh.at[pl.ds(s * RPT, RPT)],
    )
    plsc.subcore_barrier()

    def idesc(j, u):
        return pltpu.make_async_copy(idx_hbm.at[c, s, j], idx_i.at[u], isems[u])

    def gdesc(j, u, b):
        del j
        return pltpu.make_async_copy(y_hbm.at[idx_i.at[u, 0]], bufs[b], gsems[b])

    def sdesc(j, u, b):
        del j
        return pltpu.make_async_copy(bufs[b], acc_sh.at[idx_i.at[u, 1]], ssems[b])

    for t in range(3):
        idesc(t, t).start()
    idesc(0, 0).wait()
    gdesc(0, 0, 0).start()

    def body(g, carry):
        for u in range(4):
            j = g * 4 + u
            b = u % 2

            @pl.when(j >= 1)
            def _():
                sdesc(j - 1, (u - 1) % 4, 1 - b).wait()

            @pl.when(j + 1 < NCH)
            def _():
                idesc(j + 1, (u + 1) % 4).wait()
                gdesc(j + 1, (u + 1) % 4, 1 - b).start()

            gdesc(j, u, b).wait()
            sdesc(j, u, b).start(add=True)

            @pl.when(j + 3 < NCH)
            def _():
                idesc(j + 3, (u + 3) % 4).start()
        return carry

    lax.fori_loop(0, NG, body, 0)
    sdesc(NCH - 1, 3, 1).wait()
    plsc.subcore_barrier()

    @pl.when(c == 0)
    def _():
        pltpu.sync_copy(acc_sh.at[pl.ds(s * RPT, RPT)],
                        acc0_hbm.at[pl.ds(s * RPT, RPT)])

    @pl.when(c == 1)
    def _():
        pltpu.sync_copy(acc_sh.at[pl.ds(s * RPT, RPT)],
                        acc1_hbm.at[pl.ds(s * RPT, RPT)])


def _sc_mesh():
    return plsc.VectorSubcoreMesh(core_axis_name="c", subcore_axis_name="s")


def _deg_call(dst_d):
    return pl.kernel(
        _deg_body,
        out_type=jax.ShapeDtypeStruct((NC, NPAD), jnp.float32),
        mesh=_sc_mesh(),
        scratch_types=[
            pltpu.VMEM((NCHD, C), jnp.int32),
            pltpu.VMEM((C,), jnp.float32),
            pltpu.VMEM((RPT,), jnp.float32),
            pltpu.VMEM_SHARED((NPAD,), jnp.float32),
        ],
    )(dst_d)


def _edge_call(y, idx_pair):
    return pl.kernel(
        _edge_body,
        out_type=(jax.ShapeDtypeStruct((NPAD, DH), jnp.float32),
                  jax.ShapeDtypeStruct((NPAD, DH), jnp.float32)),
        mesh=_sc_mesh(),
        scratch_types=[
            pltpu.VMEM((NISLOT, 2, C), jnp.int32),
            pltpu.VMEM((C, DH), jnp.float32),
            pltpu.VMEM((C, DH), jnp.float32),
            pltpu.VMEM_SHARED((NPAD, DH), jnp.float32),
        ] + [pltpu.SemaphoreType.DMA] * (NISLOT + 4),
    )(y, idx_pair)


def _tc_a_body(x_ref, w_ref, degp_ref, dk_ref, y_ref):
    dinv = lax.rsqrt(degp_ref[0, :] + degp_ref[1, :] + 1.0)[:, None]
    xw = jnp.dot(x_ref[...] + dk_ref[0, 0], w_ref[...],
                 preferred_element_type=jnp.float32)
    y = xw * dinv
    y_ref[0] = y[:, :DH]
    y_ref[1] = y[:, DH:]


def _tc_b_body(acc0_ref, acc1_ref, degp_ref, b1_ref, w2_ref, y2_ref):
    dinv = lax.rsqrt(degp_ref[0, :] + degp_ref[1, :] + 1.0)[:, None]
    h0 = jnp.maximum(acc0_ref[...] * dinv + b1_ref[0, :], 0.0)
    h1 = jnp.maximum(acc1_ref[...] * dinv + b1_ref[1, :], 0.0)
    y2 = (jnp.dot(h0, w2_ref[:DH, :], preferred_element_type=jnp.float32)
          + jnp.dot(h1, w2_ref[DH:, :], preferred_element_type=jnp.float32))
    y2 = y2 * dinv
    y2_ref[0] = y2[:, :DH]
    y2_ref[1] = y2[:, DH:]


def _tc_c_body(acc0_ref, acc1_ref, degp_ref, b2_ref, out_ref):
    dinv = lax.rsqrt(degp_ref[0, :] + degp_ref[1, :] + 1.0)[:, None]
    h = jnp.concatenate([acc0_ref[...] * dinv, acc1_ref[...] * dinv], axis=1)
    h = jnp.maximum(h + b2_ref[0, :], 0.0)
    out_ref[...] = jnp.broadcast_to(h[:, None, :], (BN, 4, D))


def _tc_a(x, w1, degp, dk):
    return pl.pallas_call(
        _tc_a_body,
        grid=(NB,),
        in_specs=[
            pl.BlockSpec((BN, D), lambda i: (i, 0)),
            pl.BlockSpec((D, D), lambda i: (0, 0)),
            pl.BlockSpec((NC, BN), lambda i: (0, i)),
            pl.BlockSpec((1, 1), lambda i: (0, 0)),
        ],
        out_specs=pl.BlockSpec((NC, BN, DH), lambda i: (0, i, 0)),
        out_shape=jax.ShapeDtypeStruct((NC, NPAD, DH), jnp.float32),
    )(x, w1, degp, dk)


def _tc_b(acc0, acc1, degp, b1r, w2):
    return pl.pallas_call(
        _tc_b_body,
        grid=(NB,),
        in_specs=[
            pl.BlockSpec((BN, DH), lambda i: (i, 0)),
            pl.BlockSpec((BN, DH), lambda i: (i, 0)),
            pl.BlockSpec((NC, BN), lambda i: (0, i)),
            pl.BlockSpec((2, DH), lambda i: (0, 0)),
            pl.BlockSpec((D, D), lambda i: (0, 0)),
        ],
        out_specs=pl.BlockSpec((NC, BN, DH), lambda i: (0, i, 0)),
        out_shape=jax.ShapeDtypeStruct((NC, NPAD, DH), jnp.float32),
    )(acc0, acc1, degp, b1r, w2)


def _tc_c(acc0, acc1, degp, b2r):
    return pl.pallas_call(
        _tc_c_body,
        grid=(NB,),
        in_specs=[
            pl.BlockSpec((BN, DH), lambda i: (i, 0)),
            pl.BlockSpec((BN, DH), lambda i: (i, 0)),
            pl.BlockSpec((NC, BN), lambda i: (0, i)),
            pl.BlockSpec((1, D), lambda i: (0, 0)),
        ],
        out_specs=pl.BlockSpec((BN, 4, D), lambda i: (i, 0, 0)),
        out_shape=jax.ShapeDtypeStruct((N, 4, D), jnp.float32),
    )(acc0, acc1, degp, b2r)


def kernel(inputs, adj, W1, b1, W2, b2, K):
    src = adj[0].astype(jnp.int32)
    dst = adj[1].astype(jnp.int32)
    pad_idx = N + (jnp.arange(EPAD - E, dtype=jnp.int32) % (NPAD - N))
    src_p = jnp.concatenate([src, pad_idx])
    dst_p = jnp.concatenate([dst, pad_idx])
    srcs = jnp.stack([src_p, src_p + NPAD]).reshape(NC, NS, NCH, C)
    dsts = jnp.broadcast_to(dst_p.reshape(1, NS, NCH, C), (NC, NS, NCH, C))
    idx_pair = jnp.stack([srcs, dsts], axis=3)
    dst_d = dst_p.reshape(NC * NS, NCHD, C)

    dk = (jnp.asarray(K, jnp.float32) - 4.0).reshape(1, 1)
    b1r = b1.reshape(2, DH)
    b2r = b2.reshape(1, D)

    degp = _deg_call(dst_d)
    y1 = _tc_a(inputs, W1, degp, dk).reshape(NC * NPAD, DH)
    acc1a, acc1b = _edge_call(y1, idx_pair)
    y2 = _tc_b(acc1a, acc1b, degp, b1r, W2).reshape(NC * NPAD, DH)
    acc2a, acc2b = _edge_call(y2, idx_pair)
    return _tc_c(acc2a, acc2b, degp, b2r)


# speedup vs baseline: 1.5082x; 1.0242x over previous
import functools

import jax
import jax.numpy as jnp
from jax import lax
from jax.experimental import pallas as pl
from jax.experimental.pallas import tpu as pltpu
from jax.experimental.pallas import tpu_sc as plsc

N = 10000
NPAD = 10240
D = 256
DH = 128
E = 160000
EPAD = 163840
NC = 2
NS = 16
C = 128
EPT = EPAD // NS
NCH = EPT // C
EPW = EPAD // (NC * NS)
NCHD = EPW // C
RPT = NPAD // NS
BN = 2048
NB = NPAD // BN


def _deg_body(dstd_hbm, deg_hbm, dst_v, ones_v, zrow_v, deg_sh):
    c = lax.axis_index("c")
    s = lax.axis_index("s")
    w = c * NS + s
    pltpu.sync_copy(dstd_hbm.at[w], dst_v)
    for i in range(C // 16):
        ones_v[pl.ds(i * 16, 16)] = jnp.ones((16,), jnp.float32)
    for i in range(RPT // 16):
        zrow_v[pl.ds(i * 16, 16)] = jnp.zeros((16,), jnp.float32)
    pltpu.sync_copy(zrow_v, deg_sh.at[pl.ds(s * RPT, RPT)])
    plsc.subcore_barrier()

    def body(j, carry):
        pltpu.sync_copy(ones_v, deg_sh.at[dst_v.at[j]], add=True)
        return carry

    lax.fori_loop(0, NCHD, body, 0)
    plsc.subcore_barrier()
    pltpu.sync_copy(deg_sh.at[pl.ds(s * RPT, RPT)], deg_hbm.at[c, pl.ds(s * RPT, RPT)])


NISLOT = 4
NG = NCH // 4


def _edge_body(y_hbm, idx_hbm, acc0_hbm, acc1_hbm, idx_i, buf0, buf1, acc_sh,
               *sems):
    isems = sems[:NISLOT]
    gsems = sems[NISLOT:NISLOT + 2]
    ssems = sems[NISLOT + 2:]
    bufs = (buf0, buf1)
    c = lax.axis_index("c")
    s = lax.axis_index("s")
    pltpu.sync_copy(
        y_hbm.at[pl.ds(c * NPAD + s * RPT, RPT)],
        acc_sh.at[pl.ds(s * RPT, RPT)],
    )
    plsc.subcore_barrier()

    def idesc(j, u):
        return pltpu.make_async_copy(idx_hbm.at[c, s, j], idx_i.at[u], isems[u])

    def gdesc(j, u, b):
        del j
        return pltpu.make_async_copy(y_hbm.at[idx_i.at[u, 0]], bufs[b], gsems[b])

    def sdesc(j, u, b):
        del j
        return pltpu.make_async_copy(bufs[b], acc_sh.at[idx_i.at[u, 1]], ssems[b])

    for t in range(3):
        idesc(t, t).start()
    idesc(0, 0).wait()
    gdesc(0, 0, 0).start()

    def body(g, carry):
        for u in range(4):
            j = g * 4 + u
            b = u % 2

            @pl.when(j >= 1)
            def _():
                sdesc(j - 1, (u - 1) % 4, 1 - b).wait()

            @pl.when(j + 1 < NCH)
            def _():
                idesc(j + 1, (u + 1) % 4).wait()
                gdesc(j + 1, (u + 1) % 4, 1 - b).start()

            gdesc(j, u, b).wait()
            sdesc(j, u, b).start(add=True)

            @pl.when(j + 3 < NCH)
            def _():
                idesc(j + 3, (u + 3) % 4).start()
        return carry

    lax.fori_loop(0, NG, body, 0)
    sdesc(NCH - 1, 3, 1).wait()
    plsc.subcore_barrier()

    @pl.when(c == 0)
    def _():
        pltpu.sync_copy(acc_sh.at[pl.ds(s * RPT, RPT)],
                        acc0_hbm.at[pl.ds(s * RPT, RPT)])

    @pl.when(c == 1)
    def _():
        pltpu.sync_copy(acc_sh.at[pl.ds(s * RPT, RPT)],
                        acc1_hbm.at[pl.ds(s * RPT, RPT)])


def _sc_mesh():
    return plsc.VectorSubcoreMesh(core_axis_name="c", subcore_axis_name="s")


def _deg_call(dst_d):
    return pl.kernel(
        _deg_body,
        out_type=jax.ShapeDtypeStruct((NC, NPAD), jnp.float32),
        mesh=_sc_mesh(),
        scratch_types=[
            pltpu.VMEM((NCHD, C), jnp.int32),
            pltpu.VMEM((C,), jnp.float32),
            pltpu.VMEM((RPT,), jnp.float32),
            pltpu.VMEM_SHARED((NPAD,), jnp.float32),
        ],
    )(dst_d)


def _edge_call(y, idx_pair):
    return pl.kernel(
        _edge_body,
        out_type=(jax.ShapeDtypeStruct((NPAD, DH), jnp.float32),
                  jax.ShapeDtypeStruct((NPAD, DH), jnp.float32)),
        mesh=_sc_mesh(),
        scratch_types=[
            pltpu.VMEM((NISLOT, 2, C), jnp.int32),
            pltpu.VMEM((C, DH), jnp.float32),
            pltpu.VMEM((C, DH), jnp.float32),
            pltpu.VMEM_SHARED((NPAD, DH), jnp.float32),
        ] + [pltpu.SemaphoreType.DMA] * (NISLOT + 4),
    )(y, idx_pair)


def _tc_a_body(x_ref, w_ref, degp_ref, dk_ref, y_ref):
    dinv = lax.rsqrt(degp_ref[0, :] + degp_ref[1, :] + 1.0)[:, None]
    xw = jnp.dot(x_ref[...] + dk_ref[0, 0], w_ref[...],
                 preferred_element_type=jnp.float32)
    y = xw * dinv
    y_ref[0] = y[:, :DH]
    y_ref[1] = y[:, DH:]


def _tc_b_body(acc0_ref, acc1_ref, degp_ref, b1_ref, w2_ref, y2_ref):
    dinv = lax.rsqrt(degp_ref[0, :] + degp_ref[1, :] + 1.0)[:, None]
    h0 = jnp.maximum(acc0_ref[...] * dinv + b1_ref[0, :], 0.0)
    h1 = jnp.maximum(acc1_ref[...] * dinv + b1_ref[1, :], 0.0)
    y2 = (jnp.dot(h0, w2_ref[:DH, :], preferred_element_type=jnp.float32)
          + jnp.dot(h1, w2_ref[DH:, :], preferred_element_type=jnp.float32))
    y2 = y2 * dinv
    y2_ref[0] = y2[:, :DH]
    y2_ref[1] = y2[:, DH:]


def _tc_c_body(acc0_ref, acc1_ref, degp_ref, b2_ref, out_ref):
    dinv = lax.rsqrt(degp_ref[0, :] + degp_ref[1, :] + 1.0)[:, None]
    h = jnp.concatenate([acc0_ref[...] * dinv, acc1_ref[...] * dinv], axis=1)
    h = jnp.maximum(h + b2_ref[0, :], 0.0)
    out_ref[...] = jnp.broadcast_to(h[:, None, :], (BN, 4, D))


def _tc_a(x, w1, degp, dk):
    return pl.pallas_call(
        _tc_a_body,
        grid=(NB,),
        in_specs=[
            pl.BlockSpec((BN, D), lambda i: (i, 0)),
            pl.BlockSpec((D, D), lambda i: (0, 0)),
            pl.BlockSpec((NC, BN), lambda i: (0, i)),
            pl.BlockSpec((1, 1), lambda i: (0, 0)),
        ],
        out_specs=pl.BlockSpec((NC, BN, DH), lambda i: (0, i, 0)),
        out_shape=jax.ShapeDtypeStruct((NC, NPAD, DH), jnp.float32),
    )(x, w1, degp, dk)


def _tc_b(acc0, acc1, degp, b1r, w2):
    return pl.pallas_call(
        _tc_b_body,
        grid=(NB,),
        in_specs=[
            pl.BlockSpec((BN, DH), lambda i: (i, 0)),
            pl.BlockSpec((BN, DH), lambda i: (i, 0)),
            pl.BlockSpec((NC, BN), lambda i: (0, i)),
            pl.BlockSpec((2, DH), lambda i: (0, 0)),
            pl.BlockSpec((D, D), lambda i: (0, 0)),
        ],
        out_specs=pl.BlockSpec((NC, BN, DH), lambda i: (0, i, 0)),
        out_shape=jax.ShapeDtypeStruct((NC, NPAD, DH), jnp.float32),
    )(acc0, acc1, degp, b1r, w2)


def _tc_c(acc0, acc1, degp, b2r):
    return pl.pallas_call(
        _tc_c_body,
        grid=(NB,),
        in_specs=[
            pl.BlockSpec((BN, DH), lambda i: (i, 0)),
            pl.BlockSpec((BN, DH), lambda i: (i, 0)),
            pl.BlockSpec((NC, BN), lambda i: (0, i)),
            pl.BlockSpec((1, D), lambda i: (0, 0)),
        ],
        out_specs=pl.BlockSpec((BN, 4, D), lambda i: (i, 0, 0)),
        out_shape=jax.ShapeDtypeStruct((N, 4, D), jnp.float32),
    )(acc0, acc1, degp, b2r)


def kernel(inputs, adj, W1, b1, W2, b2, K):
    src = adj[0].astype(jnp.int32)
    dst = adj[1].astype(jnp.int32)
    pad_idx = N + (jnp.arange(EPAD - E, dtype=jnp.int32) % (NPAD - N))
    src_p = jnp.concatenate([src, pad_idx])
    dst_p = jnp.concatenate([dst, pad_idx])
    srcs = jnp.stack([src_p, src_p + NPAD]).reshape(NC, NS, NCH, C)
    dsts = jnp.broadcast_to(dst_p.reshape(1, NS, NCH, C), (NC, NS, NCH, C))
    idx_pair = jnp.stack([srcs, dsts], axis=3)
    dst_d = dst_p.reshape(NC * NS, NCHD, C)

    dk = (jnp.asarray(K, jnp.float32) - 4.0).reshape(1, 1)
    b1r = b1.reshape(2, DH)
    b2r = b2.reshape(1, D)

    degp = _deg_call(dst_d)
    y1 = _tc_a(inputs, W1, degp, dk).reshape(NC * NPAD, DH)
    acc1a, acc1b = _edge_call(y1, idx_pair)
    y2 = _tc_b(acc1a, acc1b, degp, b1r, W2).reshape(NC * NPAD, DH)
    acc2a, acc2b = _edge_call(y2, idx_pair)
    return _tc_c(acc2a, acc2b, degp, b2r)


# speedup vs baseline: 1.5160x; 1.0052x over previous
import functools

import jax
import jax.numpy as jnp
from jax import lax
from jax.experimental import pallas as pl
from jax.experimental.pallas import tpu as pltpu
from jax.experimental.pallas import tpu_sc as plsc

N = 10000
NPAD = 10240
D = 256
DH = 128
E = 160000
EPAD = 163840
NC = 2
NS = 16
C = 128
EPT = EPAD // NS
NCH = EPT // C
EPW = EPAD // (NC * NS)
NCHD = EPW // C
RPT = NPAD // NS
BN = 2560
NB = NPAD // BN


def _deg_body(dstd_hbm, deg_hbm, dst_v, ones_v, zrow_v, deg_sh):
    c = lax.axis_index("c")
    s = lax.axis_index("s")
    w = c * NS + s
    pltpu.sync_copy(dstd_hbm.at[w], dst_v)
    for i in range(C // 16):
        ones_v[pl.ds(i * 16, 16)] = jnp.ones((16,), jnp.float32)
    for i in range(RPT // 16):
        zrow_v[pl.ds(i * 16, 16)] = jnp.zeros((16,), jnp.float32)
    pltpu.sync_copy(zrow_v, deg_sh.at[pl.ds(s * RPT, RPT)])
    plsc.subcore_barrier()

    def body(j, carry):
        pltpu.sync_copy(ones_v, deg_sh.at[dst_v.at[j]], add=True)
        return carry

    lax.fori_loop(0, NCHD, body, 0)
    plsc.subcore_barrier()
    pltpu.sync_copy(deg_sh.at[pl.ds(s * RPT, RPT)], deg_hbm.at[c, pl.ds(s * RPT, RPT)])


NISLOT = 4
NG = NCH // 4


def _edge_body(y_hbm, idx_hbm, acc0_hbm, acc1_hbm, idx_i, buf0, buf1, acc_sh,
               *sems):
    isems = sems[:NISLOT]
    gsems = sems[NISLOT:NISLOT + 2]
    ssems = sems[NISLOT + 2:]
    bufs = (buf0, buf1)
    c = lax.axis_index("c")
    s = lax.axis_index("s")
    pltpu.sync_copy(
        y_hbm.at[pl.ds(c * NPAD + s * RPT, RPT)],
        acc_sh.at[pl.ds(s * RPT, RPT)],
    )
    plsc.subcore_barrier()

    def idesc(j, u):
        return pltpu.make_async_copy(idx_hbm.at[c, s, j], idx_i.at[u], isems[u])

    def gdesc(j, u, b):
        del j
        return pltpu.make_async_copy(y_hbm.at[idx_i.at[u, 0]], bufs[b], gsems[b])

    def sdesc(j, u, b):
        del j
        return pltpu.make_async_copy(bufs[b], acc_sh.at[idx_i.at[u, 1]], ssems[b])

    for t in range(3):
        idesc(t, t).start()
    idesc(0, 0).wait()
    gdesc(0, 0, 0).start()

    def body(g, carry):
        for u in range(4):
            j = g * 4 + u
            b = u % 2

            @pl.when(j >= 1)
            def _():
                sdesc(j - 1, (u - 1) % 4, 1 - b).wait()

            @pl.when(j + 1 < NCH)
            def _():
                idesc(j + 1, (u + 1) % 4).wait()
                gdesc(j + 1, (u + 1) % 4, 1 - b).start()

            gdesc(j, u, b).wait()
            sdesc(j, u, b).start(add=True)

            @pl.when(j + 3 < NCH)
            def _():
                idesc(j + 3, (u + 3) % 4).start()
        return carry

    lax.fori_loop(0, NG, body, 0)
    sdesc(NCH - 1, 3, 1).wait()
    plsc.subcore_barrier()

    @pl.when(c == 0)
    def _():
        pltpu.sync_copy(acc_sh.at[pl.ds(s * RPT, RPT)],
                        acc0_hbm.at[pl.ds(s * RPT, RPT)])

    @pl.when(c == 1)
    def _():
        pltpu.sync_copy(acc_sh.at[pl.ds(s * RPT, RPT)],
                        acc1_hbm.at[pl.ds(s * RPT, RPT)])


def _sc_mesh():
    return plsc.VectorSubcoreMesh(core_axis_name="c", subcore_axis_name="s")


def _deg_call(dst_d):
    return pl.kernel(
        _deg_body,
        out_type=jax.ShapeDtypeStruct((NC, NPAD), jnp.float32),
        mesh=_sc_mesh(),
        scratch_types=[
            pltpu.VMEM((NCHD, C), jnp.int32),
            pltpu.VMEM((C,), jnp.float32),
            pltpu.VMEM((RPT,), jnp.float32),
            pltpu.VMEM_SHARED((NPAD,), jnp.float32),
        ],
    )(dst_d)


def _edge_call(y, idx_pair):
    return pl.kernel(
        _edge_body,
        out_type=(jax.ShapeDtypeStruct((NPAD, DH), jnp.float32),
                  jax.ShapeDtypeStruct((NPAD, DH), jnp.float32)),
        mesh=_sc_mesh(),
        scratch_types=[
            pltpu.VMEM((NISLOT, 2, C), jnp.int32),
            pltpu.VMEM((C, DH), jnp.float32),
            pltpu.VMEM((C, DH), jnp.float32),
            pltpu.VMEM_SHARED((NPAD, DH), jnp.float32),
        ] + [pltpu.SemaphoreType.DMA] * (NISLOT + 4),
    )(y, idx_pair)


def _tc_a_body(x_ref, w_ref, degp_ref, dk_ref, y_ref):
    dinv = lax.rsqrt(degp_ref[0, :] + degp_ref[1, :] + 1.0)[:, None]
    xw = jnp.dot(x_ref[...] + dk_ref[0, 0], w_ref[...],
                 preferred_element_type=jnp.float32)
    y = xw * dinv
    y_ref[0] = y[:, :DH]
    y_ref[1] = y[:, DH:]


def _tc_b_body(acc0_ref, acc1_ref, degp_ref, b1_ref, w2_ref, y2_ref):
    dinv = lax.rsqrt(degp_ref[0, :] + degp_ref[1, :] + 1.0)[:, None]
    h0 = jnp.maximum(acc0_ref[...] * dinv + b1_ref[0, :], 0.0)
    h1 = jnp.maximum(acc1_ref[...] * dinv + b1_ref[1, :], 0.0)
    y2 = (jnp.dot(h0, w2_ref[:DH, :], preferred_element_type=jnp.float32)
          + jnp.dot(h1, w2_ref[DH:, :], preferred_element_type=jnp.float32))
    y2 = y2 * dinv
    y2_ref[0] = y2[:, :DH]
    y2_ref[1] = y2[:, DH:]


def _tc_c_body(acc0_ref, acc1_ref, degp_ref, b2_ref, out_ref):
    dinv = lax.rsqrt(degp_ref[0, :] + degp_ref[1, :] + 1.0)[:, None]
    h = jnp.concatenate([acc0_ref[...] * dinv, acc1_ref[...] * dinv], axis=1)
    h = jnp.maximum(h + b2_ref[0, :], 0.0)
    out_ref[...] = jnp.broadcast_to(h[:, None, :], (BN, 4, D))


def _tc_a(x, w1, degp, dk):
    return pl.pallas_call(
        _tc_a_body,
        grid=(NB,),
        in_specs=[
            pl.BlockSpec((BN, D), lambda i: (i, 0)),
            pl.BlockSpec((D, D), lambda i: (0, 0)),
            pl.BlockSpec((NC, BN), lambda i: (0, i)),
            pl.BlockSpec((1, 1), lambda i: (0, 0)),
        ],
        out_specs=pl.BlockSpec((NC, BN, DH), lambda i: (0, i, 0)),
        out_shape=jax.ShapeDtypeStruct((NC, NPAD, DH), jnp.float32),
    )(x, w1, degp, dk)


def _tc_b(acc0, acc1, degp, b1r, w2):
    return pl.pallas_call(
        _tc_b_body,
        grid=(NB,),
        in_specs=[
            pl.BlockSpec((BN, DH), lambda i: (i, 0)),
            pl.BlockSpec((BN, DH), lambda i: (i, 0)),
            pl.BlockSpec((NC, BN), lambda i: (0, i)),
            pl.BlockSpec((2, DH), lambda i: (0, 0)),
            pl.BlockSpec((D, D), lambda i: (0, 0)),
        ],
        out_specs=pl.BlockSpec((NC, BN, DH), lambda i: (0, i, 0)),
        out_shape=jax.ShapeDtypeStruct((NC, NPAD, DH), jnp.float32),
    )(acc0, acc1, degp, b1r, w2)


def _tc_c(acc0, acc1, degp, b2r):
    return pl.pallas_call(
        _tc_c_body,
        grid=(NB,),
        in_specs=[
            pl.BlockSpec((BN, DH), lambda i: (i, 0)),
            pl.BlockSpec((BN, DH), lambda i: (i, 0)),
            pl.BlockSpec((NC, BN), lambda i: (0, i)),
            pl.BlockSpec((1, D), lambda i: (0, 0)),
        ],
        out_specs=pl.BlockSpec((BN, 4, D), lambda i: (i, 0, 0)),
        out_shape=jax.ShapeDtypeStruct((N, 4, D), jnp.float32),
    )(acc0, acc1, degp, b2r)


def kernel(inputs, adj, W1, b1, W2, b2, K):
    src = adj[0].astype(jnp.int32)
    dst = adj[1].astype(jnp.int32)
    pad_idx = N + (jnp.arange(EPAD - E, dtype=jnp.int32) % (NPAD - N))
    src_p = jnp.concatenate([src, pad_idx])
    dst_p = jnp.concatenate([dst, pad_idx])
    srcs = jnp.stack([src_p, src_p + NPAD]).reshape(NC, NS, NCH, C)
    dsts = jnp.broadcast_to(dst_p.reshape(1, NS, NCH, C), (NC, NS, NCH, C))
    idx_pair = jnp.stack([srcs, dsts], axis=3)
    dst_d = dst_p.reshape(NC * NS, NCHD, C)

    dk = (jnp.asarray(K, jnp.float32) - 4.0).reshape(1, 1)
    b1r = b1.reshape(2, DH)
    b2r = b2.reshape(1, D)

    degp = _deg_call(dst_d)
    y1 = _tc_a(inputs, W1, degp, dk).reshape(NC * NPAD, DH)
    acc1a, acc1b = _edge_call(y1, idx_pair)
    y2 = _tc_b(acc1a, acc1b, degp, b1r, W2).reshape(NC * NPAD, DH)
    acc2a, acc2b = _edge_call(y2, idx_pair)
    return _tc_c(acc2a, acc2b, degp, b2r)


# speedup vs baseline: 1.5344x; 1.0122x over previous
import functools

import jax
import jax.numpy as jnp
from jax import lax
from jax.experimental import pallas as pl
from jax.experimental.pallas import tpu as pltpu
from jax.experimental.pallas import tpu_sc as plsc

N = 10000
NPAD = 10240
D = 256
DH = 128
E = 160000
EPAD = 163840
NC = 2
NS = 16
C = 128
EPT = EPAD // NS
NCH = EPT // C
EPW = EPAD // (NC * NS)
NCHD = EPW // C
RPT = NPAD // NS
BN = 2560
NB = NPAD // BN


def _deg_body(dstd_hbm, deg_hbm, dst_v, ones_v, zrow_v, deg_sh):
    c = lax.axis_index("c")
    s = lax.axis_index("s")
    w = c * NS + s
    pltpu.sync_copy(dstd_hbm.at[w], dst_v)
    for i in range(C // 16):
        ones_v[pl.ds(i * 16, 16)] = jnp.ones((16,), jnp.float32)
    for i in range(RPT // 16):
        zrow_v[pl.ds(i * 16, 16)] = jnp.zeros((16,), jnp.float32)
    pltpu.sync_copy(zrow_v, deg_sh.at[pl.ds(s * RPT, RPT)])
    plsc.subcore_barrier()

    def body(j, carry):
        pltpu.sync_copy(ones_v, deg_sh.at[dst_v.at[j]], add=True)
        return carry

    lax.fori_loop(0, NCHD, body, 0)
    plsc.subcore_barrier()
    pltpu.sync_copy(deg_sh.at[pl.ds(s * RPT, RPT)], deg_hbm.at[c, pl.ds(s * RPT, RPT)])


NISLOT = 4
NG = NCH // 4


def _edge_body(y_hbm, idx_hbm, acc0_hbm, acc1_hbm, idx_i, buf0, buf1, acc_sh,
               *sems):
    isems = sems[:NISLOT]
    gsems = sems[NISLOT:NISLOT + 2]
    ssems = sems[NISLOT + 2:]
    bufs = (buf0, buf1)
    c = lax.axis_index("c")
    s = lax.axis_index("s")

    def idesc(j, u):
        return pltpu.make_async_copy(idx_hbm.at[c, s, j], idx_i.at[u], isems[u])

    def gdesc(j, u, b):
        del j
        return pltpu.make_async_copy(y_hbm.at[idx_i.at[u, 0]], bufs[b], gsems[b])

    def sdesc(j, u, b):
        del j
        return pltpu.make_async_copy(bufs[b], acc_sh.at[idx_i.at[u, 1]], ssems[b])

    for t in range(3):
        idesc(t, t).start()
    pltpu.sync_copy(
        y_hbm.at[pl.ds(c * NPAD + s * RPT, RPT)],
        acc_sh.at[pl.ds(s * RPT, RPT)],
    )
    idesc(0, 0).wait()
    gdesc(0, 0, 0).start()
    plsc.subcore_barrier()

    def body(g, carry):
        for u in range(4):
            j = g * 4 + u
            b = u % 2

            @pl.when(j >= 1)
            def _():
                sdesc(j - 1, (u - 1) % 4, 1 - b).wait()

            @pl.when(j + 1 < NCH)
            def _():
                idesc(j + 1, (u + 1) % 4).wait()
                gdesc(j + 1, (u + 1) % 4, 1 - b).start()

            gdesc(j, u, b).wait()
            sdesc(j, u, b).start(add=True)

            @pl.when(j + 3 < NCH)
            def _():
                idesc(j + 3, (u + 3) % 4).start()
        return carry

    lax.fori_loop(0, NG, body, 0)
    sdesc(NCH - 1, 3, 1).wait()
    plsc.subcore_barrier()

    @pl.when(c == 0)
    def _():
        pltpu.sync_copy(acc_sh.at[pl.ds(s * RPT, RPT)],
                        acc0_hbm.at[pl.ds(s * RPT, RPT)])

    @pl.when(c == 1)
    def _():
        pltpu.sync_copy(acc_sh.at[pl.ds(s * RPT, RPT)],
                        acc1_hbm.at[pl.ds(s * RPT, RPT)])


def _sc_mesh():
    return plsc.VectorSubcoreMesh(core_axis_name="c", subcore_axis_name="s")


def _deg_call(dst_d):
    return pl.kernel(
        _deg_body,
        out_type=jax.ShapeDtypeStruct((NC, NPAD), jnp.float32),
        mesh=_sc_mesh(),
        scratch_types=[
            pltpu.VMEM((NCHD, C), jnp.int32),
            pltpu.VMEM((C,), jnp.float32),
            pltpu.VMEM((RPT,), jnp.float32),
            pltpu.VMEM_SHARED((NPAD,), jnp.float32),
        ],
    )(dst_d)


def _edge_call(y, idx_pair):
    return pl.kernel(
        _edge_body,
        out_type=(jax.ShapeDtypeStruct((NPAD, DH), jnp.float32),
                  jax.ShapeDtypeStruct((NPAD, DH), jnp.float32)),
        mesh=_sc_mesh(),
        scratch_types=[
            pltpu.VMEM((NISLOT, 2, C), jnp.int32),
            pltpu.VMEM((C, DH), jnp.float32),
            pltpu.VMEM((C, DH), jnp.float32),
            pltpu.VMEM_SHARED((NPAD, DH), jnp.float32),
        ] + [pltpu.SemaphoreType.DMA] * (NISLOT + 4),
    )(y, idx_pair)


def _tc_a_body(x_ref, w_ref, degp_ref, dk_ref, y_ref):
    dinv = lax.rsqrt(degp_ref[0, :] + degp_ref[1, :] + 1.0)[:, None]
    xw = jnp.dot(x_ref[...] + dk_ref[0, 0], w_ref[...],
                 preferred_element_type=jnp.float32)
    y = xw * dinv
    y_ref[0] = y[:, :DH]
    y_ref[1] = y[:, DH:]


def _tc_b_body(acc0_ref, acc1_ref, degp_ref, b1_ref, w2_ref, y2_ref):
    dinv = lax.rsqrt(degp_ref[0, :] + degp_ref[1, :] + 1.0)[:, None]
    h0 = jnp.maximum(acc0_ref[...] * dinv + b1_ref[0, :], 0.0)
    h1 = jnp.maximum(acc1_ref[...] * dinv + b1_ref[1, :], 0.0)
    y2 = (jnp.dot(h0, w2_ref[:DH, :], preferred_element_type=jnp.float32)
          + jnp.dot(h1, w2_ref[DH:, :], preferred_element_type=jnp.float32))
    y2 = y2 * dinv
    y2_ref[0] = y2[:, :DH]
    y2_ref[1] = y2[:, DH:]


def _tc_c_body(acc0_ref, acc1_ref, degp_ref, b2_ref, out_ref):
    dinv = lax.rsqrt(degp_ref[0, :] + degp_ref[1, :] + 1.0)[:, None]
    h = jnp.concatenate([acc0_ref[...] * dinv, acc1_ref[...] * dinv], axis=1)
    h = jnp.maximum(h + b2_ref[0, :], 0.0)
    out_ref[...] = jnp.broadcast_to(h[:, None, :], (BN, 4, D))


def _tc_a(x, w1, degp, dk):
    return pl.pallas_call(
        _tc_a_body,
        grid=(NB,),
        in_specs=[
            pl.BlockSpec((BN, D), lambda i: (i, 0)),
            pl.BlockSpec((D, D), lambda i: (0, 0)),
            pl.BlockSpec((NC, BN), lambda i: (0, i)),
            pl.BlockSpec((1, 1), lambda i: (0, 0)),
        ],
        out_specs=pl.BlockSpec((NC, BN, DH), lambda i: (0, i, 0)),
        out_shape=jax.ShapeDtypeStruct((NC, NPAD, DH), jnp.float32),
    )(x, w1, degp, dk)


def _tc_b(acc0, acc1, degp, b1r, w2):
    return pl.pallas_call(
        _tc_b_body,
        grid=(NB,),
        in_specs=[
            pl.BlockSpec((BN, DH), lambda i: (i, 0)),
            pl.BlockSpec((BN, DH), lambda i: (i, 0)),
            pl.BlockSpec((NC, BN), lambda i: (0, i)),
            pl.BlockSpec((2, DH), lambda i: (0, 0)),
            pl.BlockSpec((D, D), lambda i: (0, 0)),
        ],
        out_specs=pl.BlockSpec((NC, BN, DH), lambda i: (0, i, 0)),
        out_shape=jax.ShapeDtypeStruct((NC, NPAD, DH), jnp.float32),
    )(acc0, acc1, degp, b1r, w2)


def _tc_c(acc0, acc1, degp, b2r):
    return pl.pallas_call(
        _tc_c_body,
        grid=(NB,),
        in_specs=[
            pl.BlockSpec((BN, DH), lambda i: (i, 0)),
            pl.BlockSpec((BN, DH), lambda i: (i, 0)),
            pl.BlockSpec((NC, BN), lambda i: (0, i)),
            pl.BlockSpec((1, D), lambda i: (0, 0)),
        ],
        out_specs=pl.BlockSpec((BN, 4, D), lambda i: (i, 0, 0)),
        out_shape=jax.ShapeDtypeStruct((N, 4, D), jnp.float32),
    )(acc0, acc1, degp, b2r)


def kernel(inputs, adj, W1, b1, W2, b2, K):
    src = adj[0].astype(jnp.int32)
    dst = adj[1].astype(jnp.int32)
    pad_idx = N + (jnp.arange(EPAD - E, dtype=jnp.int32) % (NPAD - N))
    src_p = jnp.concatenate([src, pad_idx])
    dst_p = jnp.concatenate([dst, pad_idx])
    srcs = jnp.stack([src_p, src_p + NPAD]).reshape(NC, NS, NCH, C)
    dsts = jnp.broadcast_to(dst_p.reshape(1, NS, NCH, C), (NC, NS, NCH, C))
    idx_pair = jnp.stack([srcs, dsts], axis=3)
    dst_d = dst_p.reshape(NC * NS, NCHD, C)

    dk = (jnp.asarray(K, jnp.float32) - 4.0).reshape(1, 1)
    b1r = b1.reshape(2, DH)
    b2r = b2.reshape(1, D)

    degp = _deg_call(dst_d)
    y1 = _tc_a(inputs, W1, degp, dk).reshape(NC * NPAD, DH)
    acc1a, acc1b = _edge_call(y1, idx_pair)
    y2 = _tc_b(acc1a, acc1b, degp, b1r, W2).reshape(NC * NPAD, DH)
    acc2a, acc2b = _edge_call(y2, idx_pair)
    return _tc_c(acc2a, acc2b, degp, b2r)
